# Initial kernel scaffold; baseline (speedup 1.0000x reference)
import jax
import jax.numpy as jnp
from jax.experimental import pallas as pl


def kernel(c_feat, edge_idx, edge_attr, v_feat, batch_idx, params):
    raise NotImplementedError("write your pallas kernel here")


# speedup vs baseline: 3.0760x; 3.0760x over previous
import functools
import math

import jax
import jax.numpy as jnp
from jax import lax
from jax.experimental import pallas as pl
from jax.experimental.pallas import tpu as pltpu
from jax.experimental.pallas import tpu_sc as plsc

NV = 10000; NCON = 10000; NE = 160000; D = 128; B = 64
NEXP = 16; KS = 2; TOPK = 4; TEMP = 0.6
H = 4 * D

NWORK = 32
EC = 128
EPW = 5120
NEP = NWORK * EPW
NCHUNK = EPW // EC
AGG_ROWS = 10112

TB = 128
NTB = math.ceil(NV / TB)
NVP = NTB * TB


def _k0_body(cf_ref, wc_ref, bc_ref, be_ref, out_ref):
    out_ref[...] = (
        jnp.maximum(jnp.dot(cf_ref[...], wc_ref[...],
                            preferred_element_type=jnp.float32)
                    + bc_ref[...], 0.0)
        + be_ref[...]
    )


def _k0(cf8, wc8, bc2, be2):
    return pl.pallas_call(
        _k0_body,
        out_shape=jax.ShapeDtypeStruct((NCON, D), jnp.float32),
    )(cf8, wc8, bc2, be2)


def _k1_body(c2_hbm, src_hbm, dst_hbm, attr_hbm, we_hbm, zer_hbm, out_hbm,
             idx_v, dst_v, attr_v, rows_v, we_v, agg_sh, sem):
    c = lax.axis_index("c")
    s = lax.axis_index("s")
    wid = s * 2 + c

    pltpu.sync_copy(zer_hbm.at[pl.ds(0, 632)], agg_sh.at[pl.ds(s * 632, 632)])
    pltpu.sync_copy(we_hbm.at[0], we_v)
    plsc.subcore_barrier()

    we_regs = [we_v[pl.ds(16 * j, 16)] for j in range(8)]

    def chunk_body(ch, _):
        row = wid * NCHUNK + ch
        pltpu.sync_copy(src_hbm.at[row], idx_v)
        pltpu.sync_copy(dst_hbm.at[pl.ds(row, 1)], dst_v)
        pltpu.sync_copy(attr_hbm.at[row], attr_v)
        pltpu.async_copy(c2_hbm.at[idx_v], rows_v, sem).wait()

        def edge_body(i, _):
            a = plsc.load_gather(attr_v, [jnp.full((16,), i, jnp.int32)])
            for j in range(8):
                r = rows_v[i, pl.ds(16 * j, 16)]
                rows_v[i, pl.ds(16 * j, 16)] = jnp.maximum(
                    r + a * we_regs[j], 0.0)
            return 0

        lax.fori_loop(0, EC, edge_body, 0)
        pltpu.sync_copy(rows_v, agg_sh.at[dst_v.at[0]], add=True)
        return 0

    lax.fori_loop(0, NCHUNK, chunk_body, 0)
    plsc.subcore_barrier()

    pltpu.sync_copy(agg_sh.at[pl.ds(s * 632, 632)],
                    out_hbm.at[c, pl.ds(s * 632, 632)])


def _k1(c2, src2, dst2, attr2, we_row, zeros_rows):
    mesh = plsc.VectorSubcoreMesh(core_axis_name="c", subcore_axis_name="s",
                                  num_cores=2, num_subcores=16)
    f = pl.kernel(
        _k1_body,
        out_type=jax.ShapeDtypeStruct((2, NVP, D), jnp.float32),
        mesh=mesh,
        compiler_params=pltpu.CompilerParams(needs_layout_passes=False),
        scratch_types=[
            pltpu.VMEM((EC,), jnp.int32),
            pltpu.VMEM((1, EC), jnp.int32),
            pltpu.VMEM((EC,), jnp.float32),
            pltpu.VMEM((EC, D), jnp.float32),
            pltpu.VMEM((D,), jnp.float32),
            pltpu.VMEM_SHARED((AGG_ROWS, D), jnp.float32),
            pltpu.SemaphoreType.DMA,
        ],
    )
    return f(c2, src2, dst2, attr2, we_row, zeros_rows)


def _k2_body(vf_ref, wv_ref, bv_ref, a0_ref, a1_ref, wup_ref, bup_ref,
             bidx_ref, ve_ref, gp_ref, sums_scr, cnt_scr):
    t = pl.program_id(0)

    vh = jnp.maximum(jnp.dot(vf_ref[...], wv_ref[...],
                             preferred_element_type=jnp.float32)
                     + bv_ref[...], 0.0)
    agg = a0_ref[0] + a1_ref[0]
    ve = jnp.maximum(vh + jnp.dot(agg, wup_ref[...],
                                  preferred_element_type=jnp.float32)
                     + bup_ref[...], 0.0)
    ve_ref[...] = ve

    bidx = bidx_ref[:, :B]
    P = (bidx == lax.broadcasted_iota(jnp.int32, (TB, B), 1)).astype(jnp.float32)

    @pl.when(t == 0)
    def _():
        sums_scr[...] = jnp.zeros_like(sums_scr)
        cnt_scr[...] = jnp.zeros_like(cnt_scr)

    sums_scr[...] += lax.dot_general(P, ve, (((0,), (0,)), ((), ())),
                                     preferred_element_type=jnp.float32)
    cnt_scr[...] += lax.dot_general(P, jnp.ones((TB, D), jnp.float32),
                                    (((0,), (0,)), ((), ())),
                                    preferred_element_type=jnp.float32)

    @pl.when(t == pl.num_programs(0) - 1)
    def _():
        gp_ref[0] = sums_scr[...]
        gp_ref[1] = cnt_scr[...]


def _k2(vf8, wv8, bv2, aggp, wup, bup2, bidx_bc):
    grid = (NTB,)
    return pl.pallas_call(
        _k2_body,
        grid=grid,
        in_specs=[
            pl.BlockSpec((TB, 8), lambda t: (t, 0)),
            pl.BlockSpec((8, D), lambda t: (0, 0)),
            pl.BlockSpec((1, D), lambda t: (0, 0)),
            pl.BlockSpec((1, TB, D), lambda t: (0, t, 0)),
            pl.BlockSpec((1, TB, D), lambda t: (0, t, 0)),
            pl.BlockSpec((D, D), lambda t: (0, 0)),
            pl.BlockSpec((1, D), lambda t: (0, 0)),
            pl.BlockSpec((TB, D), lambda t: (t, 0)),
        ],
        out_specs=[
            pl.BlockSpec((TB, D), lambda t: (t, 0)),
            pl.BlockSpec((2, B, D), lambda t: (0, 0, 0)),
        ],
        out_shape=[
            jax.ShapeDtypeStruct((NVP, D), jnp.float32),
            jax.ShapeDtypeStruct((2, B, D), jnp.float32),
        ],
        scratch_shapes=[
            pltpu.VMEM((B, D), jnp.float32),
            pltpu.VMEM((B, D), jnp.float32),
        ],
    )(vf8, wv8, bv2, aggp[0:1], aggp[1:2], wup, bup2, bidx_bc)


def _erf(x):
    t = 1.0 / (1.0 + 0.3275911 * jnp.abs(x))
    y = 1.0 - (((((1.061405429 * t - 1.453152027) * t) + 1.421413741) * t
                - 0.284496736) * t + 0.254829592) * t * jnp.exp(-x * x)
    return jnp.sign(x) * y


def _gelu_exact(x):
    return 0.5 * x * (1.0 + lax.erf(x * (1.0 / math.sqrt(2.0))))


def _ln_rows(o, g, b):
    m = jnp.mean(o, axis=-1, keepdims=True)
    v = jnp.mean((o - m) ** 2, axis=-1, keepdims=True)
    return (o - m) * lax.rsqrt(v + 1e-5) * g + b


def _k3_body(ve_ref, gp_ref, bidx_ref,
             wg1_ref, bg1_ref, wg2_ref, bg2_ref,
             ws1_ref, bs1_ref, ws2_ref, bs2_ref, gs_ref, bsn_ref,
             wd1_ref, bd1_ref, wd2_ref, bd2_ref, gd_ref, bdn_ref,
             wh1_ref, bh1_ref, wh2t_ref, bh2_ref,
             out_ref, route_scr, acc_scr):
    t = pl.program_id(0)

    @pl.when(t == 0)
    def _():
        g_emb = gp_ref[0] / jnp.maximum(gp_ref[1], 1.0)
        h = jnp.dot(g_emb, wg1_ref[...], preferred_element_type=jnp.float32) \
            + bg1_ref[...]
        h = jnp.where(h >= 0, h, 0.2 * h)
        logits = jnp.dot(h, wg2_ref[...], preferred_element_type=jnp.float32) \
            + bg2_ref[...]
        iota = lax.broadcasted_iota(jnp.int32, (B, NEXP), 1)
        cur = logits
        mask = jnp.zeros((B, NEXP), jnp.float32)
        for _ in range(TOPK):
            m = jnp.max(cur, axis=1, keepdims=True)
            is_m = cur == m
            first = jnp.min(jnp.where(is_m, iota, NEXP), axis=1, keepdims=True)
            sel = iota == first
            mask = jnp.where(sel, 1.0, mask)
            cur = jnp.where(sel, -jnp.inf, cur)
        z = jnp.exp(logits - jnp.max(logits, axis=1, keepdims=True))
        sm = z / jnp.sum(z, axis=1, keepdims=True)
        w = sm * mask
        route_scr[...] = w / (jnp.sum(w, axis=1, keepdims=True) + 1e-12)

    x = ve_ref[...]
    bidx = bidx_ref[:, :B]
    P = (bidx == lax.broadcasted_iota(jnp.int32, (TB, B), 1)).astype(jnp.float32)
    bw = jnp.dot(P, route_scr[...], preferred_element_type=jnp.float32)

    acc = x
    for k in range(KS):
        h = jnp.dot(x, ws1_ref[k], preferred_element_type=jnp.float32) \
            + bs1_ref[k:k + 1, :]
        h = _gelu_exact(h)
        o = jnp.dot(h, ws2_ref[k], preferred_element_type=jnp.float32) \
            + bs2_ref[k:k + 1, :]
        acc = acc + (1.0 / KS) * _ln_rows(o, gs_ref[k:k + 1, :], bsn_ref[k:k + 1, :])
    acc_scr[...] = acc

    for e in range(NEXP):
        w_col = bw[:, e:e + 1]

        @pl.when(jnp.max(w_col) > 0.0)
        def _(e=e, w_col=w_col):
            h = jnp.dot(x, wd1_ref[e], preferred_element_type=jnp.float32) \
                + bd1_ref[e:e + 1, :]
            h = _gelu_exact(h)
            o = jnp.dot(h, wd2_ref[e], preferred_element_type=jnp.float32) \
                + bd2_ref[e:e + 1, :]
            acc_scr[...] += w_col * _ln_rows(o, gd_ref[e:e + 1, :],
                                             bdn_ref[e:e + 1, :])

    y = acc_scr[...]
    r = jnp.maximum(jnp.dot(y, wh1_ref[...], preferred_element_type=jnp.float32)
                    + bh1_ref[...], 0.0)
    lt = lax.dot_general(wh2t_ref[...], r, (((1,), (1,)), ((), ())),
                         preferred_element_type=jnp.float32) + bh2_ref[...]
    out_ref[...] = lt.reshape(1, 1, TB)


def _k3(ve, gpool, bidx_bc, p2):
    full = lambda shape: pl.BlockSpec(shape, lambda t: tuple(0 for _ in shape))
    grid = (NTB,)
    return pl.pallas_call(
        _k3_body,
        grid=grid,
        in_specs=[
            pl.BlockSpec((TB, D), lambda t: (t, 0)),
            full((2, B, D)),
            pl.BlockSpec((TB, D), lambda t: (t, 0)),
            full((D, D // 2)), full((1, D // 2)), full((D // 2, NEXP)), full((1, NEXP)),
            full((KS, D, H)), full((KS, H)), full((KS, H, D)), full((KS, D)),
            full((KS, D)), full((KS, D)),
            full((NEXP, D, H)), full((NEXP, H)), full((NEXP, H, D)), full((NEXP, D)),
            full((NEXP, D)), full((NEXP, D)),
            full((D, D)), full((1, D)), full((1, D)), full((1, TB)),
        ],
        out_specs=pl.BlockSpec((1, 1, TB), lambda t: (t, 0, 0)),
        out_shape=jax.ShapeDtypeStruct((NTB, 1, TB), jnp.float32),
        scratch_shapes=[
            pltpu.VMEM((B, NEXP), jnp.float32),
            pltpu.VMEM((TB, D), jnp.float32),
        ],
    )(ve, gpool, bidx_bc,
      p2['Wg1'], p2['bg1'], p2['Wg2s'], p2['bg2s'],
      p2['Ws1'], p2['bs1'], p2['Ws2'], p2['bs2'], p2['gs'], p2['bs'],
      p2['Wd1'], p2['bd1'], p2['Wd2'], p2['bd2'], p2['gd'], p2['bd'],
      p2['Wh1'], p2['bh1'], p2['Wh2t'], p2['bh2'])


def kernel(c_feat, edge_idx, edge_attr, v_feat, batch_idx, params):
    p = params
    f32 = jnp.float32

    cf8 = jnp.pad(c_feat.astype(f32), ((0, 0), (0, 4)))
    wc8 = jnp.pad(p['Wc'].astype(f32), ((0, 4), (0, 0)))
    vf8 = jnp.pad(v_feat.astype(f32), ((0, NVP - NV), (0, 2)))
    wv8 = jnp.pad(p['Wv'].astype(f32), ((0, 2), (0, 0)))

    npad = NEP - NE
    ar = jnp.arange(npad, dtype=jnp.int32)
    src_p = jnp.concatenate([edge_idx[0].astype(jnp.int32), (ar * 97) % NCON])
    dst_p = jnp.concatenate([edge_idx[1].astype(jnp.int32), NV + (ar % 16)])
    attr_p = jnp.concatenate([edge_attr[:, 0].astype(f32),
                              jnp.zeros((npad,), f32)])
    src2 = src_p.reshape(NEP // EC, EC)
    dst2 = dst_p.reshape(NEP // EC, EC)
    attr2 = attr_p.reshape(NEP // EC, EC)
    we_row = p['We'].astype(f32).reshape(1, D)
    zeros_rows = jnp.zeros((640, D), f32)

    bidx_bc = jnp.broadcast_to(
        jnp.pad(batch_idx.astype(jnp.int32), (0, NVP - NV),
                constant_values=B)[:, None], (NVP, D))

    scale = (p['alpha'].astype(f32) / TEMP)
    p2 = {
        'Wg1': p['Wg1'].astype(f32), 'bg1': p['bg1'].astype(f32).reshape(1, -1),
        'Wg2s': p['Wg2'].astype(f32) * scale,
        'bg2s': (p['bg2'].astype(f32) * scale
                 + p['ebias'].astype(f32)).reshape(1, -1),
        'Ws1': p['Ws1'].astype(f32), 'bs1': p['bs1'].astype(f32),
        'Ws2': p['Ws2'].astype(f32), 'bs2': p['bs2'].astype(f32),
        'gs': p['gs'].astype(f32), 'bs': p['bs'].astype(f32),
        'Wd1': p['Wd1'].astype(f32), 'bd1': p['bd1'].astype(f32),
        'Wd2': p['Wd2'].astype(f32), 'bd2': p['bd2'].astype(f32),
        'gd': p['gd'].astype(f32), 'bd': p['bd'].astype(f32),
        'Wh1': p['Wh1'].astype(f32), 'bh1': p['bh1'].astype(f32).reshape(1, -1),
        'Wh2t': p['Wh2'].astype(f32).reshape(1, D),
        'bh2': jnp.broadcast_to(p['bh2'].astype(f32).reshape(1, 1), (1, TB)),
    }

    c2 = _k0(cf8, wc8, p['bc'].astype(f32).reshape(1, D),
             p['be'].astype(f32).reshape(1, D))
    aggp = _k1(c2, src2, dst2, attr2, we_row, zeros_rows)
    ve, gpool = _k2(vf8, wv8, p['bv'].astype(f32).reshape(1, D), aggp,
                    p['Wup'].astype(f32),
                    p['bup'].astype(f32).reshape(1, D), bidx_bc)
    out3 = _k3(ve, gpool, bidx_bc, p2)
    return out3.reshape(NVP)[:NV]


# speedup vs baseline: 3.3179x; 1.0786x over previous
import functools
import math

import jax
import jax.numpy as jnp
from jax import lax
from jax.experimental import pallas as pl
from jax.experimental.pallas import tpu as pltpu
from jax.experimental.pallas import tpu_sc as plsc

NV = 10000; NCON = 10000; NE = 160000; D = 128; B = 64
NEXP = 16; KS = 2; TOPK = 4; TEMP = 0.6
H = 4 * D

NWORK = 32
EC = 128
EPW = 5120
NEP = NWORK * EPW
NCHUNK = EPW // EC
AGG_ROWS = 10112

TB = 128
NTB = math.ceil(NV / TB)
NVP = NTB * TB


def _k0_body(cf_ref, wc_ref, bc_ref, be_ref, out_ref):
    out_ref[...] = (
        jnp.maximum(jnp.dot(cf_ref[...], wc_ref[...],
                            preferred_element_type=jnp.float32)
                    + bc_ref[...], 0.0)
        + be_ref[...]
    )


def _k0(cf8, wc8, bc2, be2):
    return pl.pallas_call(
        _k0_body,
        out_shape=jax.ShapeDtypeStruct((NCON, D), jnp.float32),
    )(cf8, wc8, bc2, be2)


def _k1_body(c2_hbm, src_hbm, dst_hbm, attr_hbm, we_hbm, zer_hbm, out_hbm,
             idx2_v, dst2_v, attr2_v, rows2_v, we_v, agg_sh, sem0, sem1):
    c = lax.axis_index("c")
    s = lax.axis_index("s")
    wid = s * 2 + c
    sems = (sem0, sem1)

    pltpu.sync_copy(zer_hbm.at[pl.ds(0, 632)], agg_sh.at[pl.ds(s * 632, 632)])
    pltpu.sync_copy(we_hbm.at[0], we_v)
    plsc.subcore_barrier()

    we_regs = [we_v[pl.ds(16 * j, 16)] for j in range(8)]

    def prefetch(ch, b):
        row = wid * NCHUNK + ch
        pltpu.sync_copy(src_hbm.at[row], idx2_v.at[b])
        pltpu.sync_copy(dst_hbm.at[row], dst2_v.at[b])
        pltpu.sync_copy(attr_hbm.at[row], attr2_v.at[b])
        pltpu.async_copy(c2_hbm.at[idx2_v.at[b]], rows2_v.at[b], sems[b])

    def process(ch, b):
        @pl.when(ch + 1 < NCHUNK)
        def _():
            prefetch(ch + 1, 1 - b)

        pltpu.make_async_copy(c2_hbm.at[idx2_v.at[b]], rows2_v.at[b],
                              sems[b]).wait()

        def edge_body(i, _):
            a = plsc.load_gather(
                attr2_v, [jnp.full((16,), b, jnp.int32),
                          jnp.full((16,), i, jnp.int32)])
            for j in range(8):
                r = rows2_v[b, i, pl.ds(16 * j, 16)]
                rows2_v[b, i, pl.ds(16 * j, 16)] = jnp.maximum(
                    r + a * we_regs[j], 0.0)
            return 0

        lax.fori_loop(0, EC, edge_body, 0)
        pltpu.sync_copy(rows2_v.at[b], agg_sh.at[dst2_v.at[b]], add=True)

    prefetch(0, 0)

    def pair_body(pr, _):
        process(2 * pr, 0)
        process(2 * pr + 1, 1)
        return 0

    lax.fori_loop(0, NCHUNK // 2, pair_body, 0)
    plsc.subcore_barrier()

    pltpu.sync_copy(agg_sh.at[pl.ds(s * 632, 632)],
                    out_hbm.at[c, pl.ds(s * 632, 632)])


def _k1(c2, src2, dst2, attr2, we_row, zeros_rows):
    mesh = plsc.VectorSubcoreMesh(core_axis_name="c", subcore_axis_name="s",
                                  num_cores=2, num_subcores=16)
    f = pl.kernel(
        _k1_body,
        out_type=jax.ShapeDtypeStruct((2, NVP, D), jnp.float32),
        mesh=mesh,
        compiler_params=pltpu.CompilerParams(needs_layout_passes=False),
        scratch_types=[
            pltpu.VMEM((2, EC), jnp.int32),
            pltpu.VMEM((2, EC), jnp.int32),
            pltpu.VMEM((2, EC), jnp.float32),
            pltpu.VMEM((2, EC, D), jnp.float32),
            pltpu.VMEM((D,), jnp.float32),
            pltpu.VMEM_SHARED((AGG_ROWS, D), jnp.float32),
            pltpu.SemaphoreType.DMA,
            pltpu.SemaphoreType.DMA,
        ],
    )
    return f(c2, src2, dst2, attr2, we_row, zeros_rows)


def _k2_body(vf_ref, wv_ref, bv_ref, a0_ref, a1_ref, wup_ref, bup_ref,
             bidx_ref, ve_ref, gp_ref, sums_scr, cnt_scr):
    t = pl.program_id(0)

    vh = jnp.maximum(jnp.dot(vf_ref[...], wv_ref[...],
                             preferred_element_type=jnp.float32)
                     + bv_ref[...], 0.0)
    agg = a0_ref[0] + a1_ref[0]
    ve = jnp.maximum(vh + jnp.dot(agg, wup_ref[...],
                                  preferred_element_type=jnp.float32)
                     + bup_ref[...], 0.0)
    ve_ref[...] = ve

    bidx = bidx_ref[:, :B]
    P = (bidx == lax.broadcasted_iota(jnp.int32, (TB, B), 1)).astype(jnp.float32)

    @pl.when(t == 0)
    def _():
        sums_scr[...] = jnp.zeros_like(sums_scr)
        cnt_scr[...] = jnp.zeros_like(cnt_scr)

    sums_scr[...] += lax.dot_general(P, ve, (((0,), (0,)), ((), ())),
                                     preferred_element_type=jnp.float32)
    cnt_scr[...] += lax.dot_general(P, jnp.ones((TB, D), jnp.float32),
                                    (((0,), (0,)), ((), ())),
                                    preferred_element_type=jnp.float32)

    @pl.when(t == pl.num_programs(0) - 1)
    def _():
        gp_ref[0] = sums_scr[...]
        gp_ref[1] = cnt_scr[...]


def _k2(vf8, wv8, bv2, aggp, wup, bup2, bidx_bc):
    grid = (NTB,)
    return pl.pallas_call(
        _k2_body,
        grid=grid,
        in_specs=[
            pl.BlockSpec((TB, 8), lambda t: (t, 0)),
            pl.BlockSpec((8, D), lambda t: (0, 0)),
            pl.BlockSpec((1, D), lambda t: (0, 0)),
            pl.BlockSpec((1, TB, D), lambda t: (0, t, 0)),
            pl.BlockSpec((1, TB, D), lambda t: (0, t, 0)),
            pl.BlockSpec((D, D), lambda t: (0, 0)),
            pl.BlockSpec((1, D), lambda t: (0, 0)),
            pl.BlockSpec((TB, D), lambda t: (t, 0)),
        ],
        out_specs=[
            pl.BlockSpec((TB, D), lambda t: (t, 0)),
            pl.BlockSpec((2, B, D), lambda t: (0, 0, 0)),
        ],
        out_shape=[
            jax.ShapeDtypeStruct((NVP, D), jnp.float32),
            jax.ShapeDtypeStruct((2, B, D), jnp.float32),
        ],
        scratch_shapes=[
            pltpu.VMEM((B, D), jnp.float32),
            pltpu.VMEM((B, D), jnp.float32),
        ],
    )(vf8, wv8, bv2, aggp[0:1], aggp[1:2], wup, bup2, bidx_bc)


def _erf(x):
    t = 1.0 / (1.0 + 0.3275911 * jnp.abs(x))
    y = 1.0 - (((((1.061405429 * t - 1.453152027) * t) + 1.421413741) * t
                - 0.284496736) * t + 0.254829592) * t * jnp.exp(-x * x)
    return jnp.sign(x) * y


def _gelu2(x):
    return x * (1.0 + lax.erf(x * (1.0 / math.sqrt(2.0))))


def _ln_rows(o, g, b):
    m = jnp.mean(o, axis=-1, keepdims=True)
    v = jnp.mean((o - m) ** 2, axis=-1, keepdims=True)
    return (o - m) * lax.rsqrt(v + 1e-5) * g + b


def _k3_body(ve_ref, gp_ref, bidx_ref,
             wg1_ref, bg1_ref, wg2_ref, bg2_ref,
             ws1_ref, bs1_ref, ws2_ref, bs2_ref, gs_ref, bsn_ref,
             wd1_ref, bd1_ref, wd2_ref, bd2_ref, gd_ref, bdn_ref,
             wh1_ref, bh1_ref, wh2t_ref, bh2_ref,
             out_ref, route_scr, acc_scr):
    t = pl.program_id(0)

    @pl.when(t == 0)
    def _():
        g_emb = gp_ref[0] / jnp.maximum(gp_ref[1], 1.0)
        h = jnp.dot(g_emb, wg1_ref[...], preferred_element_type=jnp.float32) \
            + bg1_ref[...]
        h = jnp.where(h >= 0, h, 0.2 * h)
        logits = jnp.dot(h, wg2_ref[...], preferred_element_type=jnp.float32) \
            + bg2_ref[...]
        iota = lax.broadcasted_iota(jnp.int32, (B, NEXP), 1)
        cur = logits
        mask = jnp.zeros((B, NEXP), jnp.float32)
        for _ in range(TOPK):
            m = jnp.max(cur, axis=1, keepdims=True)
            is_m = cur == m
            first = jnp.min(jnp.where(is_m, iota, NEXP), axis=1, keepdims=True)
            sel = iota == first
            mask = jnp.where(sel, 1.0, mask)
            cur = jnp.where(sel, -jnp.inf, cur)
        z = jnp.exp(logits - jnp.max(logits, axis=1, keepdims=True))
        sm = z / jnp.sum(z, axis=1, keepdims=True)
        w = sm * mask
        route_scr[...] = w / (jnp.sum(w, axis=1, keepdims=True) + 1e-12)

    x = ve_ref[...]
    bidx = bidx_ref[:, :B]
    P = (bidx == lax.broadcasted_iota(jnp.int32, (TB, B), 1)).astype(jnp.float32)
    bw = jnp.dot(P, route_scr[...], preferred_element_type=jnp.float32)

    acc = x
    for k in range(KS):
        h = jnp.dot(x, ws1_ref[k], preferred_element_type=jnp.float32) \
            + bs1_ref[k:k + 1, :]
        h = _gelu2(h)
        o = jnp.dot(h, ws2_ref[k], preferred_element_type=jnp.float32) \
            + bs2_ref[k:k + 1, :]
        acc = acc + (1.0 / KS) * _ln_rows(o, gs_ref[k:k + 1, :], bsn_ref[k:k + 1, :])
    acc_scr[...] = acc

    for e in range(NEXP):
        w_col = bw[:, e:e + 1]

        @pl.when(jnp.max(w_col) > 0.0)
        def _(e=e, w_col=w_col):
            h = jnp.dot(x, wd1_ref[e], preferred_element_type=jnp.float32) \
                + bd1_ref[e:e + 1, :]
            h = _gelu2(h)
            o = jnp.dot(h, wd2_ref[e], preferred_element_type=jnp.float32) \
                + bd2_ref[e:e + 1, :]
            acc_scr[...] += w_col * _ln_rows(o, gd_ref[e:e + 1, :],
                                             bdn_ref[e:e + 1, :])

    y = acc_scr[...]
    r = jnp.maximum(jnp.dot(y, wh1_ref[...], preferred_element_type=jnp.float32)
                    + bh1_ref[...], 0.0)
    lt = lax.dot_general(wh2t_ref[...], r, (((1,), (1,)), ((), ())),
                         preferred_element_type=jnp.float32) + bh2_ref[...]
    out_ref[...] = lt.reshape(1, 1, TB)


def _k3(ve, gpool, bidx_bc, p2):
    full = lambda shape: pl.BlockSpec(shape, lambda t: tuple(0 for _ in shape))
    grid = (NTB,)
    return pl.pallas_call(
        _k3_body,
        grid=grid,
        in_specs=[
            pl.BlockSpec((TB, D), lambda t: (t, 0)),
            full((2, B, D)),
            pl.BlockSpec((TB, D), lambda t: (t, 0)),
            full((D, D // 2)), full((1, D // 2)), full((D // 2, NEXP)), full((1, NEXP)),
            full((KS, D, H)), full((KS, H)), full((KS, H, D)), full((KS, D)),
            full((KS, D)), full((KS, D)),
            full((NEXP, D, H)), full((NEXP, H)), full((NEXP, H, D)), full((NEXP, D)),
            full((NEXP, D)), full((NEXP, D)),
            full((D, D)), full((1, D)), full((1, D)), full((1, TB)),
        ],
        out_specs=pl.BlockSpec((1, 1, TB), lambda t: (t, 0, 0)),
        out_shape=jax.ShapeDtypeStruct((NTB, 1, TB), jnp.float32),
        scratch_shapes=[
            pltpu.VMEM((B, NEXP), jnp.float32),
            pltpu.VMEM((TB, D), jnp.float32),
        ],
    )(ve, gpool, bidx_bc,
      p2['Wg1'], p2['bg1'], p2['Wg2s'], p2['bg2s'],
      p2['Ws1'], p2['bs1'], p2['Ws2'], p2['bs2'], p2['gs'], p2['bs'],
      p2['Wd1'], p2['bd1'], p2['Wd2'], p2['bd2'], p2['gd'], p2['bd'],
      p2['Wh1'], p2['bh1'], p2['Wh2t'], p2['bh2'])


def kernel(c_feat, edge_idx, edge_attr, v_feat, batch_idx, params):
    p = params
    f32 = jnp.float32

    cf8 = jnp.pad(c_feat.astype(f32), ((0, 0), (0, 4)))
    wc8 = jnp.pad(p['Wc'].astype(f32), ((0, 4), (0, 0)))
    vf8 = jnp.pad(v_feat.astype(f32), ((0, NVP - NV), (0, 2)))
    wv8 = jnp.pad(p['Wv'].astype(f32), ((0, 2), (0, 0)))

    npad = NEP - NE
    ar = jnp.arange(npad, dtype=jnp.int32)
    src_p = jnp.concatenate([edge_idx[0].astype(jnp.int32), (ar * 97) % NCON])
    dst_p = jnp.concatenate([edge_idx[1].astype(jnp.int32), NV + (ar % 16)])
    attr_p = jnp.concatenate([edge_attr[:, 0].astype(f32),
                              jnp.zeros((npad,), f32)])
    src2 = src_p.reshape(NEP // EC, EC)
    dst2 = dst_p.reshape(NEP // EC, EC)
    attr2 = attr_p.reshape(NEP // EC, EC)
    we_row = p['We'].astype(f32).reshape(1, D)
    zeros_rows = jnp.zeros((640, D), f32)

    bidx_bc = jnp.broadcast_to(
        jnp.pad(batch_idx.astype(jnp.int32), (0, NVP - NV),
                constant_values=B)[:, None], (NVP, D))

    scale = (p['alpha'].astype(f32) / TEMP)
    p2 = {
        'Wg1': p['Wg1'].astype(f32), 'bg1': p['bg1'].astype(f32).reshape(1, -1),
        'Wg2s': p['Wg2'].astype(f32) * scale,
        'bg2s': (p['bg2'].astype(f32) * scale
                 + p['ebias'].astype(f32)).reshape(1, -1),
        'Ws1': p['Ws1'].astype(f32), 'bs1': p['bs1'].astype(f32),
        'Ws2': p['Ws2'].astype(f32) * 0.5, 'bs2': p['bs2'].astype(f32),
        'gs': p['gs'].astype(f32), 'bs': p['bs'].astype(f32),
        'Wd1': p['Wd1'].astype(f32), 'bd1': p['bd1'].astype(f32),
        'Wd2': p['Wd2'].astype(f32) * 0.5, 'bd2': p['bd2'].astype(f32),
        'gd': p['gd'].astype(f32), 'bd': p['bd'].astype(f32),
        'Wh1': p['Wh1'].astype(f32), 'bh1': p['bh1'].astype(f32).reshape(1, -1),
        'Wh2t': p['Wh2'].astype(f32).reshape(1, D),
        'bh2': jnp.broadcast_to(p['bh2'].astype(f32).reshape(1, 1), (1, TB)),
    }

    c2 = _k0(cf8, wc8, p['bc'].astype(f32).reshape(1, D),
             p['be'].astype(f32).reshape(1, D))
    aggp = _k1(c2, src2, dst2, attr2, we_row, zeros_rows)
    ve, gpool = _k2(vf8, wv8, p['bv'].astype(f32).reshape(1, D), aggp,
                    p['Wup'].astype(f32),
                    p['bup'].astype(f32).reshape(1, D), bidx_bc)
    out3 = _k3(ve, gpool, bidx_bc, p2)
    return out3.reshape(NVP)[:NV]


# speedup vs baseline: 4.6757x; 1.4092x over previous
import functools
import math

import jax
import jax.numpy as jnp
from jax import lax
from jax.experimental import pallas as pl
from jax.experimental.pallas import tpu as pltpu
from jax.experimental.pallas import tpu_sc as plsc

NV = 10000; NCON = 10000; NE = 160000; D = 128; B = 64
NEXP = 16; KS = 2; TOPK = 4; TEMP = 0.6
H = 4 * D

NWORK = 32
EC = 128
EPW = 5120
NEP = NWORK * EPW
NCHUNK = EPW // EC

TB = 256
NTB = math.ceil(NV / TB)
NVP = NTB * TB
AGG_ROWS = NVP
ZCH = NVP // 16


def _k0_body(cf_ref, wc_ref, bc_ref, be_ref, out_ref):
    out_ref[...] = (
        jnp.maximum(jnp.dot(cf_ref[...], wc_ref[...],
                            preferred_element_type=jnp.float32)
                    + bc_ref[...], 0.0)
        + be_ref[...]
    )


def _k0(cf8, wc8, bc2, be2):
    return pl.pallas_call(
        _k0_body,
        out_shape=jax.ShapeDtypeStruct((NCON, D), jnp.float32),
    )(cf8, wc8, bc2, be2)


def _k1_body(c2_hbm, src_hbm, dst_hbm, attr_hbm, we_hbm, zer_hbm, out_hbm,
             idx2_v, dst2_v, attr2_v, rows2_v, we_v, agg_sh, sem0, sem1):
    c = lax.axis_index("c")
    s = lax.axis_index("s")
    wid = s * 2 + c
    sems = (sem0, sem1)

    pltpu.sync_copy(zer_hbm.at[pl.ds(0, ZCH)], agg_sh.at[pl.ds(s * ZCH, ZCH)])
    pltpu.sync_copy(we_hbm.at[0], we_v)
    plsc.subcore_barrier()

    we_regs = [we_v[pl.ds(16 * j, 16)] for j in range(8)]

    def prefetch(ch, b):
        row = wid * NCHUNK + ch
        pltpu.sync_copy(src_hbm.at[row], idx2_v.at[b])
        pltpu.sync_copy(dst_hbm.at[row], dst2_v.at[b])
        pltpu.sync_copy(attr_hbm.at[row], attr2_v.at[b])
        pltpu.async_copy(c2_hbm.at[idx2_v.at[b]], rows2_v.at[b], sems[b])

    def process(ch, b):
        @pl.when(ch + 1 < NCHUNK)
        def _():
            prefetch(ch + 1, 1 - b)

        pltpu.make_async_copy(c2_hbm.at[idx2_v.at[b]], rows2_v.at[b],
                              sems[b]).wait()

        def edge_body(i, _):
            a = plsc.load_gather(
                attr2_v, [jnp.full((16,), b, jnp.int32),
                          jnp.full((16,), i, jnp.int32)])
            for j in range(8):
                r = rows2_v[b, i, pl.ds(16 * j, 16)]
                rows2_v[b, i, pl.ds(16 * j, 16)] = jnp.maximum(
                    r + a * we_regs[j], 0.0)
            return 0

        lax.fori_loop(0, EC, edge_body, 0, unroll=4)
        pltpu.sync_copy(rows2_v.at[b], agg_sh.at[dst2_v.at[b]], add=True)

    prefetch(0, 0)

    def pair_body(pr, _):
        process(2 * pr, 0)
        process(2 * pr + 1, 1)
        return 0

    lax.fori_loop(0, NCHUNK // 2, pair_body, 0)
    plsc.subcore_barrier()

    pltpu.sync_copy(agg_sh.at[pl.ds(s * ZCH, ZCH)],
                    out_hbm.at[c, pl.ds(s * ZCH, ZCH)])


def _k1(c2, src2, dst2, attr2, we_row, zeros_rows):
    mesh = plsc.VectorSubcoreMesh(core_axis_name="c", subcore_axis_name="s",
                                  num_cores=2, num_subcores=16)
    f = pl.kernel(
        _k1_body,
        out_type=jax.ShapeDtypeStruct((2, NVP, D), jnp.float32),
        mesh=mesh,
        compiler_params=pltpu.CompilerParams(needs_layout_passes=False),
        scratch_types=[
            pltpu.VMEM((2, EC), jnp.int32),
            pltpu.VMEM((2, EC), jnp.int32),
            pltpu.VMEM((2, EC), jnp.float32),
            pltpu.VMEM((2, EC, D), jnp.float32),
            pltpu.VMEM((D,), jnp.float32),
            pltpu.VMEM_SHARED((AGG_ROWS, D), jnp.float32),
            pltpu.SemaphoreType.DMA,
            pltpu.SemaphoreType.DMA,
        ],
    )
    return f(c2, src2, dst2, attr2, we_row, zeros_rows)


def _k2_body(vf_ref, wv_ref, bv_ref, a0_ref, a1_ref, wup_ref, bup_ref,
             bidx_ref, ve_ref, gp_ref, sums_scr, cnt_scr):
    t = pl.program_id(0)

    vh = jnp.maximum(jnp.dot(vf_ref[...], wv_ref[...],
                             preferred_element_type=jnp.float32)
                     + bv_ref[...], 0.0)
    agg = a0_ref[0] + a1_ref[0]
    ve = jnp.maximum(vh + jnp.dot(agg, wup_ref[...],
                                  preferred_element_type=jnp.float32)
                     + bup_ref[...], 0.0)
    ve_ref[...] = ve

    bidx = bidx_ref[:, :B]
    P = (bidx == lax.broadcasted_iota(jnp.int32, (TB, B), 1)).astype(jnp.float32)

    @pl.when(t == 0)
    def _():
        sums_scr[...] = jnp.zeros_like(sums_scr)
        cnt_scr[...] = jnp.zeros_like(cnt_scr)

    sums_scr[...] += lax.dot_general(P, ve, (((0,), (0,)), ((), ())),
                                     preferred_element_type=jnp.float32)
    cnt_scr[...] += lax.dot_general(P, jnp.ones((TB, D), jnp.float32),
                                    (((0,), (0,)), ((), ())),
                                    preferred_element_type=jnp.float32)

    @pl.when(t == pl.num_programs(0) - 1)
    def _():
        gp_ref[0] = sums_scr[...]
        gp_ref[1] = cnt_scr[...]


def _k2(vf8, wv8, bv2, aggp, wup, bup2, bidx_bc):
    grid = (NTB,)
    return pl.pallas_call(
        _k2_body,
        grid=grid,
        in_specs=[
            pl.BlockSpec((TB, 8), lambda t: (t, 0)),
            pl.BlockSpec((8, D), lambda t: (0, 0)),
            pl.BlockSpec((1, D), lambda t: (0, 0)),
            pl.BlockSpec((1, TB, D), lambda t: (0, t, 0)),
            pl.BlockSpec((1, TB, D), lambda t: (0, t, 0)),
            pl.BlockSpec((D, D), lambda t: (0, 0)),
            pl.BlockSpec((1, D), lambda t: (0, 0)),
            pl.BlockSpec((TB, D), lambda t: (t, 0)),
        ],
        out_specs=[
            pl.BlockSpec((TB, D), lambda t: (t, 0)),
            pl.BlockSpec((2, B, D), lambda t: (0, 0, 0)),
        ],
        out_shape=[
            jax.ShapeDtypeStruct((NVP, D), jnp.float32),
            jax.ShapeDtypeStruct((2, B, D), jnp.float32),
        ],
        scratch_shapes=[
            pltpu.VMEM((B, D), jnp.float32),
            pltpu.VMEM((B, D), jnp.float32),
        ],
    )(vf8, wv8, bv2, aggp[0:1], aggp[1:2], wup, bup2, bidx_bc)


def _erf(x):
    t = 1.0 / (1.0 + 0.3275911 * jnp.abs(x))
    y = 1.0 - (((((1.061405429 * t - 1.453152027) * t) + 1.421413741) * t
                - 0.284496736) * t + 0.254829592) * t * jnp.exp(-x * x)
    return jnp.sign(x) * y


def _gelu2(x):
    return x * (1.0 + lax.erf(x * (1.0 / math.sqrt(2.0))))


def _ln_rows(o, g, b):
    m = jnp.mean(o, axis=-1, keepdims=True)
    v = jnp.mean((o - m) ** 2, axis=-1, keepdims=True)
    return (o - m) * lax.rsqrt(v + 1e-5) * g + b


def _k3_body(ve_ref, gp_ref, bidx_ref,
             wg1_ref, bg1_ref, wg2_ref, bg2_ref,
             ws1_ref, bs1_ref, ws2_ref, bs2_ref, gs_ref, bsn_ref,
             wd1_ref, bd1_ref, wd2_ref, bd2_ref, gd_ref, bdn_ref,
             wh1_ref, bh1_ref, wh2t_ref, bh2_ref,
             out_ref, route_scr, acc_scr):
    t = pl.program_id(0)

    @pl.when(t == 0)
    def _():
        g_emb = gp_ref[0] / jnp.maximum(gp_ref[1], 1.0)
        h = jnp.dot(g_emb, wg1_ref[...], preferred_element_type=jnp.float32) \
            + bg1_ref[...]
        h = jnp.where(h >= 0, h, 0.2 * h)
        logits = jnp.dot(h, wg2_ref[...], preferred_element_type=jnp.float32) \
            + bg2_ref[...]
        iota = lax.broadcasted_iota(jnp.int32, (B, NEXP), 1)
        cur = logits
        mask = jnp.zeros((B, NEXP), jnp.float32)
        for _ in range(TOPK):
            m = jnp.max(cur, axis=1, keepdims=True)
            is_m = cur == m
            first = jnp.min(jnp.where(is_m, iota, NEXP), axis=1, keepdims=True)
            sel = iota == first
            mask = jnp.where(sel, 1.0, mask)
            cur = jnp.where(sel, -jnp.inf, cur)
        z = jnp.exp(logits - jnp.max(logits, axis=1, keepdims=True))
        sm = z / jnp.sum(z, axis=1, keepdims=True)
        w = sm * mask
        route_scr[...] = w / (jnp.sum(w, axis=1, keepdims=True) + 1e-12)

    x = ve_ref[...]
    bidx = bidx_ref[:, :B]
    P = (bidx == lax.broadcasted_iota(jnp.int32, (TB, B), 1)).astype(jnp.float32)
    bw = jnp.dot(P, route_scr[...], preferred_element_type=jnp.float32)

    acc = x
    for k in range(KS):
        h = jnp.dot(x, ws1_ref[k], preferred_element_type=jnp.float32) \
            + bs1_ref[k:k + 1, :]
        h = _gelu2(h)
        o = jnp.dot(h, ws2_ref[k], preferred_element_type=jnp.float32) \
            + bs2_ref[k:k + 1, :]
        acc = acc + (1.0 / KS) * _ln_rows(o, gs_ref[k:k + 1, :], bsn_ref[k:k + 1, :])
    acc_scr[...] = acc

    for e in range(NEXP):
        w_col = bw[:, e:e + 1]

        @pl.when(jnp.max(w_col) > 0.0)
        def _(e=e, w_col=w_col):
            h = jnp.dot(x, wd1_ref[e], preferred_element_type=jnp.float32) \
                + bd1_ref[e:e + 1, :]
            h = _gelu2(h)
            o = jnp.dot(h, wd2_ref[e], preferred_element_type=jnp.float32) \
                + bd2_ref[e:e + 1, :]
            acc_scr[...] += w_col * _ln_rows(o, gd_ref[e:e + 1, :],
                                             bdn_ref[e:e + 1, :])

    y = acc_scr[...]
    r = jnp.maximum(jnp.dot(y, wh1_ref[...], preferred_element_type=jnp.float32)
                    + bh1_ref[...], 0.0)
    lt = lax.dot_general(wh2t_ref[...], r, (((1,), (1,)), ((), ())),
                         preferred_element_type=jnp.float32) + bh2_ref[...]
    out_ref[...] = lt.reshape(1, 1, TB)


def _k3(ve, gpool, bidx_bc, p2):
    full = lambda shape: pl.BlockSpec(shape, lambda t: tuple(0 for _ in shape))
    grid = (NTB,)
    return pl.pallas_call(
        _k3_body,
        grid=grid,
        in_specs=[
            pl.BlockSpec((TB, D), lambda t: (t, 0)),
            full((2, B, D)),
            pl.BlockSpec((TB, D), lambda t: (t, 0)),
            full((D, D // 2)), full((1, D // 2)), full((D // 2, NEXP)), full((1, NEXP)),
            full((KS, D, H)), full((KS, H)), full((KS, H, D)), full((KS, D)),
            full((KS, D)), full((KS, D)),
            full((NEXP, D, H)), full((NEXP, H)), full((NEXP, H, D)), full((NEXP, D)),
            full((NEXP, D)), full((NEXP, D)),
            full((D, D)), full((1, D)), full((1, D)), full((1, TB)),
        ],
        out_specs=pl.BlockSpec((1, 1, TB), lambda t: (t, 0, 0)),
        out_shape=jax.ShapeDtypeStruct((NTB, 1, TB), jnp.float32),
        scratch_shapes=[
            pltpu.VMEM((B, NEXP), jnp.float32),
            pltpu.VMEM((TB, D), jnp.float32),
        ],
    )(ve, gpool, bidx_bc,
      p2['Wg1'], p2['bg1'], p2['Wg2s'], p2['bg2s'],
      p2['Ws1'], p2['bs1'], p2['Ws2'], p2['bs2'], p2['gs'], p2['bs'],
      p2['Wd1'], p2['bd1'], p2['Wd2'], p2['bd2'], p2['gd'], p2['bd'],
      p2['Wh1'], p2['bh1'], p2['Wh2t'], p2['bh2'])


def kernel(c_feat, edge_idx, edge_attr, v_feat, batch_idx, params):
    p = params
    f32 = jnp.float32

    cf8 = jnp.pad(c_feat.astype(f32), ((0, 0), (0, 4)))
    wc8 = jnp.pad(p['Wc'].astype(f32), ((0, 4), (0, 0)))
    vf8 = jnp.pad(v_feat.astype(f32), ((0, NVP - NV), (0, 2)))
    wv8 = jnp.pad(p['Wv'].astype(f32), ((0, 2), (0, 0)))

    npad = NEP - NE
    ar = jnp.arange(npad, dtype=jnp.int32)
    src_p = jnp.concatenate([edge_idx[0].astype(jnp.int32), (ar * 97) % NCON])
    dst_p = jnp.concatenate([edge_idx[1].astype(jnp.int32), NV + (ar % 16)])
    attr_p = jnp.concatenate([edge_attr[:, 0].astype(f32),
                              jnp.zeros((npad,), f32)])
    src2 = src_p.reshape(NEP // EC, EC)
    dst2 = dst_p.reshape(NEP // EC, EC)
    attr2 = attr_p.reshape(NEP // EC, EC)
    we_row = p['We'].astype(f32).reshape(1, D)
    zeros_rows = jnp.zeros((ZCH, D), f32)

    bidx_bc = jnp.broadcast_to(
        jnp.pad(batch_idx.astype(jnp.int32), (0, NVP - NV),
                constant_values=B)[:, None], (NVP, D))

    scale = (p['alpha'].astype(f32) / TEMP)
    p2 = {
        'Wg1': p['Wg1'].astype(f32), 'bg1': p['bg1'].astype(f32).reshape(1, -1),
        'Wg2s': p['Wg2'].astype(f32) * scale,
        'bg2s': (p['bg2'].astype(f32) * scale
                 + p['ebias'].astype(f32)).reshape(1, -1),
        'Ws1': p['Ws1'].astype(f32), 'bs1': p['bs1'].astype(f32),
        'Ws2': p['Ws2'].astype(f32) * 0.5, 'bs2': p['bs2'].astype(f32),
        'gs': p['gs'].astype(f32), 'bs': p['bs'].astype(f32),
        'Wd1': p['Wd1'].astype(f32), 'bd1': p['bd1'].astype(f32),
        'Wd2': p['Wd2'].astype(f32) * 0.5, 'bd2': p['bd2'].astype(f32),
        'gd': p['gd'].astype(f32), 'bd': p['bd'].astype(f32),
        'Wh1': p['Wh1'].astype(f32), 'bh1': p['bh1'].astype(f32).reshape(1, -1),
        'Wh2t': p['Wh2'].astype(f32).reshape(1, D),
        'bh2': jnp.broadcast_to(p['bh2'].astype(f32).reshape(1, 1), (1, TB)),
    }

    c2 = _k0(cf8, wc8, p['bc'].astype(f32).reshape(1, D),
             p['be'].astype(f32).reshape(1, D))
    aggp = _k1(c2, src2, dst2, attr2, we_row, zeros_rows)
    ve, gpool = _k2(vf8, wv8, p['bv'].astype(f32).reshape(1, D), aggp,
                    p['Wup'].astype(f32),
                    p['bup'].astype(f32).reshape(1, D), bidx_bc)
    out3 = _k3(ve, gpool, bidx_bc, p2)
    return out3.reshape(NVP)[:NV]
